# Initial kernel scaffold; baseline (speedup 1.0000x reference)
import jax
import jax.numpy as jnp
from jax.experimental import pallas as pl


def kernel(x, edge_index, edge_weight, W1z, b1z, W1r, b1r, W1h, b1h, W2z, b2z, W2r, b2r, W2h, b2h):
    raise NotImplementedError("write your pallas kernel here")


# speedup vs baseline: 2.2653x; 2.2653x over previous
import functools

import jax
import jax.numpy as jnp
from jax import lax
from jax.experimental import pallas as pl
from jax.experimental.pallas import tpu as pltpu
from jax.experimental.pallas import tpu_sc as plsc

N = 10000
NP = 10240
E = 320000
EP = 327680
D = 128
NC = 2
NS = 16
EDGES_PER_TILE = EP // NS
EC = 320
NSPLIT = 5120
NA = 5248
AZ_PER_TILE = NA // NS
AO_PER_TILE = NSPLIT // NS
PCHUNK = 2048
CAPB = EDGES_PER_TILE + 2 * EC
TREG = 2 * CAPB
PART = NS * TREG

_MESH = plsc.VectorSubcoreMesh(
    core_axis_name="c", subcore_axis_name="s", num_cores=NC, num_subcores=NS)


def _prep_body(src_hbm, dst_hbm, w_hbm,
               pg_o, ps_o, pn_o, pg_i, ps_i, pn_i, cnts_hbm,
               deg_o_s, deg_i_s,
               degb, srcb, dstb, wb, nob, posb, cntv, padv, padi,
               padpl, padph):
    c = lax.axis_index("c")
    s = lax.axis_index("s")
    iota = lax.iota(jnp.int32, 16)
    zero16 = jnp.zeros((16,), jnp.float32)
    izero16 = jnp.zeros((16,), jnp.int32)

    def zb(i, _):
        nob[pl.ds(i * 16, 16)] = zero16
        return 0
    lax.fori_loop(0, PCHUNK // 16, zb, 0)

    def zp(i, _):
        padv[pl.ds(i * 16, 16)] = zero16
        padi[pl.ds(i * 16, 16)] = izero16
        return 0
    lax.fori_loop(0, (2 * EC) // 16, zp, 0)

    pltpu.sync_copy(nob.at[pl.ds(0, 640)], deg_o_s.at[pl.ds(s * 640, 640)])
    pltpu.sync_copy(nob.at[pl.ds(0, 640)], deg_i_s.at[pl.ds(s * 640, 640)])
    plsc.subcore_barrier()

    def acc(k, _):
        off = s * EDGES_PER_TILE + k * PCHUNK
        pltpu.sync_copy(src_hbm.at[pl.ds(off, PCHUNK)], srcb)
        pltpu.sync_copy(dst_hbm.at[pl.ds(off, PCHUNK)], dstb)
        pltpu.sync_copy(w_hbm.at[pl.ds(off, PCHUNK)], wb)
        pltpu.sync_copy(wb, deg_o_s.at[srcb], add=True)
        pltpu.sync_copy(wb, deg_i_s.at[dstb], add=True)
        return 0
    lax.fori_loop(0, EDGES_PER_TILE // PCHUNK, acc, 0)
    plsc.subcore_barrier()

    base_lo = s * TREG
    base_hi = s * TREG + CAPB

    def ppass(deg_s, key_is_dst, pg, ps, pn):
        def pchunk(k, carry):
            lo_pos, hi_pos = carry
            off = s * EDGES_PER_TILE + k * PCHUNK
            pltpu.sync_copy(src_hbm.at[pl.ds(off, PCHUNK)], srcb)
            pltpu.sync_copy(dst_hbm.at[pl.ds(off, PCHUNK)], dstb)
            pltpu.sync_copy(w_hbm.at[pl.ds(off, PCHUNK)], wb)
            if key_is_dst:
                pltpu.sync_copy(deg_s.at[srcb], degb)
            else:
                pltpu.sync_copy(deg_s.at[dstb], degb)

            def grp(g, pcarry):
                lo, hi = pcarry
                sl = pl.ds(g * 16, 16)
                s16 = srcb[sl]
                d16 = dstb[sl]
                w16 = wb[sl]
                nob[sl] = jnp.where(w16 > 0.0, w16 / degb[sl], 0.0)
                key = d16 if key_is_dst else s16
                cond = key < NSPLIT
                inc = jnp.where(cond, jnp.full((16,), 1, jnp.int32),
                                jnp.full((16,), 0, jnp.int32))
                for sh in (1, 2, 4, 8):
                    rolled = jnp.take(inc, jnp.maximum(iota - sh, 0))
                    inc = inc + jnp.where(iota >= sh, rolled,
                                          jnp.full((16,), 0, jnp.int32))
                tot = jnp.take(inc, jnp.full((16,), 15, jnp.int32))
                pos = jnp.where(cond, lo + inc - 1,
                                hi + (iota + 1 - inc) - 1)
                posb[sl] = pos
                return (lo + tot, hi + 16 - tot)
            lo_pos, hi_pos = lax.fori_loop(0, PCHUNK // 16, grp,
                                           (lo_pos, hi_pos))
            if key_is_dst:
                pltpu.sync_copy(srcb, pg.at[posb])
                pltpu.sync_copy(dstb, ps.at[posb])
            else:
                pltpu.sync_copy(dstb, pg.at[posb])
                pltpu.sync_copy(srcb, ps.at[posb])
            pltpu.sync_copy(nob, pn.at[posb])
            return (lo_pos, hi_pos)

        lo_pos, hi_pos = lax.fori_loop(
            0, EDGES_PER_TILE // PCHUNK, pchunk,
            (jnp.full((16,), base_lo, jnp.int32),
             jnp.full((16,), base_hi, jnp.int32)))

        def wpad(i, _):
            sl = pl.ds(i * 16, 16)
            padpl[sl] = lo_pos + i * 16 + iota
            padph[sl] = hi_pos + i * 16 + iota
            return 0
        lax.fori_loop(0, (2 * EC) // 16, wpad, 0)
        pltpu.sync_copy(padi, pg.at[padpl])
        pltpu.sync_copy(padi, ps.at[padpl])
        pltpu.sync_copy(padv, pn.at[padpl])
        pltpu.sync_copy(padi, pg.at[padph])
        pltpu.sync_copy(padi, ps.at[padph])
        pltpu.sync_copy(padv, pn.at[padph])

        cntv[...] = lo_pos - base_lo
        pltpu.sync_copy(cntv.at[pl.ds(0, 8)],
                        cnts_hbm.at[pl.ds((c * NS + s) * 8, 8)])

    @pl.when(c == 0)
    def _():
        ppass(deg_o_s, True, pg_o, ps_o, pn_o)

    @pl.when(c == 1)
    def _():
        ppass(deg_i_s, False, pg_i, ps_i, pn_i)


_prep = pl.kernel(
    _prep_body,
    out_type=[pltpu.MemorySpace.HBM((PART,), jnp.int32),
              pltpu.MemorySpace.HBM((PART,), jnp.int32),
              pltpu.MemorySpace.HBM((PART,), jnp.float32),
              pltpu.MemorySpace.HBM((PART,), jnp.int32),
              pltpu.MemorySpace.HBM((PART,), jnp.int32),
              pltpu.MemorySpace.HBM((PART,), jnp.float32),
              pltpu.MemorySpace.HBM((NC * NS * 8,), jnp.int32)],
    mesh=_MESH,
    scratch_types=[
        pltpu.VMEM_SHARED((NP,), jnp.float32),
        pltpu.VMEM_SHARED((NP,), jnp.float32),
        pltpu.VMEM((PCHUNK,), jnp.float32),
        pltpu.VMEM((PCHUNK,), jnp.int32),
        pltpu.VMEM((PCHUNK,), jnp.int32),
        pltpu.VMEM((PCHUNK,), jnp.float32),
        pltpu.VMEM((PCHUNK,), jnp.float32),
        pltpu.VMEM((PCHUNK,), jnp.int32),
        pltpu.VMEM((16,), jnp.int32),
        pltpu.VMEM((2 * EC,), jnp.float32),
        pltpu.VMEM((2 * EC,), jnp.int32),
        pltpu.VMEM((2 * EC,), jnp.int32),
        pltpu.VMEM((2 * EC,), jnp.int32),
    ],
)


def _cell_body(zeros_hbm, x_hbm, pg_o, ps_o, pn_o, pg_i, ps_i, pn_i,
               cnts_hbm,
               txo_hbm, txi_hbm,
               acc,
               rows0, rows1, g0, s0, n0, l0, g1, s1, n1, l1, cntb, cnt_sh,
               cnt_sm,
               sg0, sg1, ss0, ss1, si0, si1):
    c = lax.axis_index("c")
    s = lax.axis_index("s")
    iota = lax.iota(jnp.int32, 16)

    rows = (rows0, rows1)
    gb = (g0, g1)
    sb = (s0, s1)
    nb = (n0, n1)
    lb = (l0, l1)
    sg = (sg0, sg1)
    ss = (ss0, ss1)
    si = (si0, si1)

    pltpu.sync_copy(cnts_hbm.at[pl.ds((c * NS + s) * 8, 8)],
                    cntb.at[pl.ds(0, 8)])
    pltpu.sync_copy(cntb.at[pl.ds(0, 8)], cnt_sh.at[pl.ds(s * 8, 8)])
    pltpu.sync_copy(cnt_sh.at[pl.ds(s * 8, 8)], cnt_sm)
    cnt_lo = cnt_sm[0]

    def run(gather_hbm, scat_hbm, nrm_hbm, out_hbm):
        def issue_idx(k, b, base):
            pltpu.async_copy(gather_hbm.at[pl.ds(base + k * EC, EC)],
                             gb[b], si[b])
            pltpu.async_copy(scat_hbm.at[pl.ds(base + k * EC, EC)],
                             sb[b], si[b])
            pltpu.async_copy(nrm_hbm.at[pl.ds(base + k * EC, EC)],
                             nb[b], si[b])

        def wait_idx(b):
            pltpu.make_async_copy(gather_hbm.at[pl.ds(0, EC)], gb[b],
                                  si[b]).wait()
            pltpu.make_async_copy(scat_hbm.at[pl.ds(0, EC)], sb[b],
                                  si[b]).wait()
            pltpu.make_async_copy(nrm_hbm.at[pl.ds(0, EC)], nb[b],
                                  si[b]).wait()

        def issue_gather(b):
            pltpu.async_copy(x_hbm.at[gb[b]], rows[b], sg[b])

        def wait_gather(b):
            pltpu.make_async_copy(x_hbm.at[pl.ds(0, EC)], rows[b],
                                  sg[b]).wait()

        def issue_scatter(b):
            pltpu.async_copy(rows[b], acc.at[lb[b]], ss[b], add=True)

        def wait_scatter(b):
            pltpu.make_async_copy(rows[b], acc.at[pl.ds(0, EC)],
                                  ss[b]).wait()

        def compute(b, half):
            @plsc.parallel_loop(0, EC // 16, 1, unroll=2)
            def grp(g):
                gsl = pl.ds(g * 16, 16)
                n16 = nb[b][gsl]
                d16 = sb[b][gsl]
                t16 = iota + (g % 8) * 16
                if half == 0:
                    cond = d16 < NSPLIT
                    rel = d16
                else:
                    cond = d16 >= NSPLIT
                    rel = d16 - NSPLIT
                lb[b][gsl] = jnp.where(cond, rel, NSPLIT + t16)
                for j in range(16):
                    e = g * 16 + j
                    bc = jnp.take(n16, jnp.full((16,), j, jnp.int32))
                    for q in range(D // 16):
                        qsl = pl.ds(q * 16, 16)
                        rows[b][e, qsl] = rows[b][e, qsl] * bc

        for half in range(2):
            pltpu.sync_copy(zeros_hbm, acc.at[pl.ds(s * AZ_PER_TILE,
                                                    AZ_PER_TILE)])
            plsc.subcore_barrier()

            base = s * TREG + half * CAPB
            nedge = jnp.where(half == 0, cnt_lo, EDGES_PER_TILE - cnt_lo)
            trip = 2 * ((nedge + 2 * EC - 1) // (2 * EC))

            def step(k, b):
                @pl.when(k < trip)
                def _():
                    @pl.when(k + 1 < trip)
                    def _():
                        @pl.when(k >= 1)
                        def _():
                            wait_scatter(1 - b)
                        wait_idx(1 - b)
                        issue_gather(1 - b)
                    wait_gather(b)
                    compute(b, half)
                    issue_scatter(b)
                    @pl.when(k + 2 < trip)
                    def _():
                        issue_idx(k + 2, b, base)

            @pl.when(trip > 0)
            def _():
                issue_idx(0, 0, base)
                issue_idx(1, 1, base)
                wait_idx(0)
                issue_gather(0)

            def pair(t, _):
                step(2 * t, 0)
                step(2 * t + 1, 1)
                return 0
            lax.fori_loop(0, trip // 2, pair, 0)
            @pl.when(trip > 0)
            def _():
                wait_scatter(0)
                wait_scatter(1)
            plsc.subcore_barrier()
            pltpu.sync_copy(
                acc.at[pl.ds(s * AO_PER_TILE, AO_PER_TILE)],
                out_hbm.at[pl.ds(half * NSPLIT + s * AO_PER_TILE,
                                 AO_PER_TILE)])
            plsc.subcore_barrier()

    @pl.when(c == 0)
    def _():
        run(pg_o, ps_o, pn_o, txo_hbm)

    @pl.when(c == 1)
    def _():
        run(pg_i, ps_i, pn_i, txi_hbm)


_diffuse = pl.kernel(
    _cell_body,
    out_type=[pltpu.MemorySpace.HBM((NP, D), jnp.float32),
              pltpu.MemorySpace.HBM((NP, D), jnp.float32)],
    mesh=_MESH,
    scratch_types=[
        pltpu.VMEM_SHARED((NA, D), jnp.float32),
        pltpu.VMEM((EC, D), jnp.float32),
        pltpu.VMEM((EC, D), jnp.float32),
        pltpu.VMEM((EC,), jnp.int32),
        pltpu.VMEM((EC,), jnp.int32),
        pltpu.VMEM((EC,), jnp.float32),
        pltpu.VMEM((EC,), jnp.int32),
        pltpu.VMEM((EC,), jnp.int32),
        pltpu.VMEM((EC,), jnp.int32),
        pltpu.VMEM((EC,), jnp.float32),
        pltpu.VMEM((EC,), jnp.int32),
        pltpu.VMEM((16,), jnp.int32),
        pltpu.VMEM_SHARED((NS * 8,), jnp.int32),
        pltpu.SMEM((8,), jnp.int32),
        pltpu.SemaphoreType.DMA,
        pltpu.SemaphoreType.DMA,
        pltpu.SemaphoreType.DMA,
        pltpu.SemaphoreType.DMA,
        pltpu.SemaphoreType.DMA,
        pltpu.SemaphoreType.DMA,
    ],
)


def _dense_body(x_ref, to_ref, ti_ref, wz_ref, wh_ref, bz_ref, bh_ref,
                out_ref, *, relu):
    g = jnp.concatenate([x_ref[...], to_ref[...], ti_ref[...]], axis=1)
    z = jax.nn.sigmoid(
        jnp.dot(g, wz_ref[...], preferred_element_type=jnp.float32)
        + bz_ref[...])
    ht = jnp.tanh(
        jnp.dot(g, wh_ref[...], preferred_element_type=jnp.float32)
        + bh_ref[...])
    o = (1.0 - z) * ht
    if relu:
        o = jnp.maximum(o, 0.0)
    out_ref[...] = o


def _dense(x, to, ti, wz, wh, bz, bh, relu):
    blk = 2000
    grid = (N // blk,)
    row_spec = pl.BlockSpec((blk, D), lambda i: (i, 0))
    full_spec = pl.BlockSpec((3 * D, D), lambda i: (0, 0))
    bias_spec = pl.BlockSpec((1, D), lambda i: (0, 0))
    return pl.pallas_call(
        functools.partial(_dense_body, relu=relu),
        out_shape=jax.ShapeDtypeStruct((N, D), jnp.float32),
        grid=grid,
        in_specs=[row_spec, row_spec, row_spec, full_spec, full_spec,
                  bias_spec, bias_spec],
        out_specs=row_spec,
    )(x, to, ti, wz, wh, bz, bh)


def _gate_weights(W):
    a = W[0, 0][:D] + W[1, 0][:D]
    return jnp.concatenate([a, W[0, 1][:D], W[1, 1][:D]], axis=0)


def kernel(x, edge_index, edge_weight, W1z, b1z, W1r, b1r, W1h, b1h,
           W2z, b2z, W2r, b2r, W2h, b2h):
    f32 = jnp.float32
    src = edge_index[0].astype(jnp.int32)
    dst = edge_index[1].astype(jnp.int32)
    w = edge_weight.astype(f32)
    pad = EP - E
    srcp = jnp.concatenate([src, jnp.zeros((pad,), jnp.int32)])
    dstp = jnp.concatenate([dst, jnp.zeros((pad,), jnp.int32)])
    wp = jnp.concatenate([w, jnp.zeros((pad,), f32)])

    pg_o, ps_o, pn_o, pg_i, ps_i, pn_i, cnts = _prep(srcp, dstp, wp)
    zrows = jnp.zeros((AZ_PER_TILE, D), f32)

    wz1 = _gate_weights(W1z)
    wh1 = _gate_weights(W1h)
    wz2 = _gate_weights(W2z)
    wh2 = _gate_weights(W2h)
    bz1 = b1z.reshape(1, D)
    bh1 = b1h.reshape(1, D)
    bz2 = b2z.reshape(1, D)
    bh2 = b2h.reshape(1, D)

    xf = x.astype(f32)
    to1, ti1 = _diffuse(zrows, xf, pg_o, ps_o, pn_o, pg_i, ps_i, pn_i, cnts)
    h1 = _dense(xf, to1[:N], ti1[:N], wz1, wh1, bz1, bh1, relu=True)
    to2, ti2 = _diffuse(zrows, h1, pg_o, ps_o, pn_o, pg_i, ps_i, pn_i, cnts)
    h2 = _dense(h1, to2[:N], ti2[:N], wz2, wh2, bz2, bh2, relu=False)
    return h2


# speedup vs baseline: 5.3234x; 2.3500x over previous
import functools

import jax
import jax.numpy as jnp
from jax import lax
from jax.experimental import pallas as pl
from jax.experimental.pallas import tpu as pltpu
from jax.experimental.pallas import tpu_sc as plsc

N = 10000
NP = 10240
E = 320000
EP = 327680
D = 128
DH = 64
NC = 2
NS = 16
ROWS_PER_TILE = NP // NS
EDGES_PER_TILE = EP // NS
EC = 320
N_CHUNKS = EDGES_PER_TILE // EC
NSPLIT = 5120
NA = 5248
AZ_PER_TILE = NA // NS
AO_PER_TILE = NSPLIT // NS
NORM_CHUNK = 2048
NORM_EDGES_PER_TILE = EP // (NC * NS)

_MESH = plsc.VectorSubcoreMesh(
    core_axis_name="c", subcore_axis_name="s", num_cores=NC, num_subcores=NS)


def _norm_body(src_hbm, dst_hbm, w_hbm, no_hbm, ni_hbm,
               deg_o_s, deg_i_s, deg_o_l, deg_i_l,
               srcb, dstb, wb, nob, nib):
    c = lax.axis_index("c")
    s = lax.axis_index("s")
    zero16 = jnp.zeros((16,), jnp.float32)

    def zb(i, _):
        nob[pl.ds(i * 16, 16)] = zero16
        return 0
    lax.fori_loop(0, NORM_CHUNK // 16, zb, 0)

    pltpu.sync_copy(nob.at[pl.ds(0, 640)], deg_o_s.at[pl.ds(s * 640, 640)])
    pltpu.sync_copy(nob.at[pl.ds(0, 640)], deg_i_s.at[pl.ds(s * 640, 640)])
    plsc.subcore_barrier()

    def acc(k, _):
        off = s * EDGES_PER_TILE + k * NORM_CHUNK
        pltpu.sync_copy(src_hbm.at[pl.ds(off, NORM_CHUNK)], srcb)
        pltpu.sync_copy(dst_hbm.at[pl.ds(off, NORM_CHUNK)], dstb)
        pltpu.sync_copy(w_hbm.at[pl.ds(off, NORM_CHUNK)], wb)
        pltpu.sync_copy(wb, deg_o_s.at[srcb], add=True)
        pltpu.sync_copy(wb, deg_i_s.at[dstb], add=True)
        return 0
    lax.fori_loop(0, EDGES_PER_TILE // NORM_CHUNK, acc, 0)
    plsc.subcore_barrier()

    def nrm(k, _):
        off = (c * NS + s) * NORM_EDGES_PER_TILE + k * NORM_CHUNK
        pltpu.sync_copy(src_hbm.at[pl.ds(off, NORM_CHUNK)], srcb)
        pltpu.sync_copy(dst_hbm.at[pl.ds(off, NORM_CHUNK)], dstb)
        pltpu.sync_copy(w_hbm.at[pl.ds(off, NORM_CHUNK)], wb)
        pltpu.sync_copy(deg_o_s.at[srcb], deg_o_l)
        pltpu.sync_copy(deg_i_s.at[dstb], deg_i_l)

        def grp2(g, _):
            sl = pl.ds(g * 16, 16)
            w16 = wb[sl]
            nob[sl] = jnp.where(w16 > 0.0, w16 / deg_o_l[sl], 0.0)
            nib[sl] = jnp.where(w16 > 0.0, w16 / deg_i_l[sl], 0.0)
            return 0
        lax.fori_loop(0, NORM_CHUNK // 16, grp2, 0)
        pltpu.sync_copy(nob, no_hbm.at[pl.ds(off, NORM_CHUNK)])
        pltpu.sync_copy(nib, ni_hbm.at[pl.ds(off, NORM_CHUNK)])
        return 0
    lax.fori_loop(0, NORM_EDGES_PER_TILE // NORM_CHUNK, nrm, 0)


_norms = pl.kernel(
    _norm_body,
    out_type=[pltpu.MemorySpace.HBM((EP,), jnp.float32),
              pltpu.MemorySpace.HBM((EP,), jnp.float32)],
    mesh=_MESH,
    scratch_types=[
        pltpu.VMEM_SHARED((NP,), jnp.float32),
        pltpu.VMEM_SHARED((NP,), jnp.float32),
        pltpu.VMEM((NORM_CHUNK,), jnp.float32),
        pltpu.VMEM((NORM_CHUNK,), jnp.float32),
        pltpu.VMEM((NORM_CHUNK,), jnp.int32),
        pltpu.VMEM((NORM_CHUNK,), jnp.int32),
        pltpu.VMEM((NORM_CHUNK,), jnp.float32),
        pltpu.VMEM((NORM_CHUNK,), jnp.float32),
        pltpu.VMEM((NORM_CHUNK,), jnp.float32),
    ],
)


def _cell_body(zeros_hbm, x_hbm, src_hbm, dst_hbm, no_hbm, ni_hbm,
               txo_hbm, txi_hbm,
               acc,
               rows0, rows1, g0, s0, n0, l0, g1, s1, n1, l1,
               sg0, sg1, ss0, ss1, si0, si1):
    c = lax.axis_index("c")
    s = lax.axis_index("s")
    iota = lax.iota(jnp.int32, 16)
    ebase = s * EDGES_PER_TILE

    rows = (rows0, rows1)
    gb = (g0, g1)
    sb = (s0, s1)
    nb = (n0, n1)
    lb = (l0, l1)
    sg = (sg0, sg1)
    ss = (ss0, ss1)
    si = (si0, si1)

    def run(gather_hbm, scat_hbm, nrm_hbm, out_hbm):
        def issue_idx(k, b):
            pltpu.async_copy(gather_hbm.at[pl.ds(ebase + k * EC, EC)],
                             gb[b], si[b])
            pltpu.async_copy(scat_hbm.at[pl.ds(ebase + k * EC, EC)],
                             sb[b], si[b])
            pltpu.async_copy(nrm_hbm.at[pl.ds(ebase + k * EC, EC)],
                             nb[b], si[b])

        def wait_idx(b):
            pltpu.make_async_copy(gather_hbm.at[pl.ds(0, EC)], gb[b],
                                  si[b]).wait()
            pltpu.make_async_copy(scat_hbm.at[pl.ds(0, EC)], sb[b],
                                  si[b]).wait()
            pltpu.make_async_copy(nrm_hbm.at[pl.ds(0, EC)], nb[b],
                                  si[b]).wait()

        def issue_gather(b):
            pltpu.async_copy(x_hbm.at[gb[b]], rows[b], sg[b])

        def wait_gather(b):
            pltpu.make_async_copy(x_hbm.at[pl.ds(0, EC)], rows[b],
                                  sg[b]).wait()

        def issue_scatter(b):
            pltpu.async_copy(rows[b], acc.at[lb[b]], ss[b], add=True)

        def wait_scatter(b):
            pltpu.make_async_copy(rows[b], acc.at[pl.ds(0, EC)],
                                  ss[b]).wait()

        def compute(b, half):
            @plsc.parallel_loop(0, EC // 16, 1, unroll=2)
            def grp(g):
                gsl = pl.ds(g * 16, 16)
                n16 = nb[b][gsl]
                d16 = sb[b][gsl]
                t16 = iota + (g % 8) * 16
                if half == 0:
                    cond = d16 < NSPLIT
                    rel = d16
                else:
                    cond = d16 >= NSPLIT
                    rel = d16 - NSPLIT
                lb[b][gsl] = jnp.where(cond, rel, NSPLIT + t16)
                for j in range(16):
                    e = g * 16 + j
                    bc = jnp.take(n16, jnp.full((16,), j, jnp.int32))
                    for q in range(D // 16):
                        qsl = pl.ds(q * 16, 16)
                        rows[b][e, qsl] = rows[b][e, qsl] * bc

        def step(k, b, half):
            @pl.when(k + 1 < N_CHUNKS)
            def _():
                @pl.when(k >= 1)
                def _():
                    wait_scatter(1 - b)
                wait_idx(1 - b)
                issue_gather(1 - b)
            wait_gather(b)
            compute(b, half)
            issue_scatter(b)
            @pl.when(k + 2 < N_CHUNKS)
            def _():
                issue_idx_dyn(k + 2, b)

        def issue_idx_dyn(k, b):
            pltpu.async_copy(gather_hbm.at[pl.ds(ebase + k * EC, EC)],
                             gb[b], si[b])
            pltpu.async_copy(scat_hbm.at[pl.ds(ebase + k * EC, EC)],
                             sb[b], si[b])
            pltpu.async_copy(nrm_hbm.at[pl.ds(ebase + k * EC, EC)],
                             nb[b], si[b])

        for half in range(2):
            pltpu.sync_copy(zeros_hbm, acc.at[pl.ds(s * AZ_PER_TILE,
                                                    AZ_PER_TILE)])
            plsc.subcore_barrier()

            issue_idx(0, 0)
            issue_idx(1, 1)
            wait_idx(0)
            issue_gather(0)

            def pair(t, _):
                step(2 * t, 0, half)
                step(2 * t + 1, 1, half)
                return 0
            lax.fori_loop(0, N_CHUNKS // 2, pair, 0)
            wait_scatter(0)
            wait_scatter(1)
            plsc.subcore_barrier()
            pltpu.sync_copy(
                acc.at[pl.ds(s * AO_PER_TILE, AO_PER_TILE)],
                out_hbm.at[pl.ds(half * NSPLIT + s * AO_PER_TILE,
                                 AO_PER_TILE)])
            plsc.subcore_barrier()

    @pl.when(c == 0)
    def _():
        run(src_hbm, dst_hbm, no_hbm, txo_hbm)

    @pl.when(c == 1)
    def _():
        run(dst_hbm, src_hbm, ni_hbm, txi_hbm)


_diffuse = pl.kernel(
    _cell_body,
    out_type=[pltpu.MemorySpace.HBM((NP, D), jnp.float32),
              pltpu.MemorySpace.HBM((NP, D), jnp.float32)],
    mesh=_MESH,
    scratch_types=[
        pltpu.VMEM_SHARED((NA, D), jnp.float32),
        pltpu.VMEM((EC, D), jnp.float32),
        pltpu.VMEM((EC, D), jnp.float32),
        pltpu.VMEM((EC,), jnp.int32),
        pltpu.VMEM((EC,), jnp.int32),
        pltpu.VMEM((EC,), jnp.float32),
        pltpu.VMEM((EC,), jnp.int32),
        pltpu.VMEM((EC,), jnp.int32),
        pltpu.VMEM((EC,), jnp.int32),
        pltpu.VMEM((EC,), jnp.float32),
        pltpu.VMEM((EC,), jnp.int32),
        pltpu.SemaphoreType.DMA,
        pltpu.SemaphoreType.DMA,
        pltpu.SemaphoreType.DMA,
        pltpu.SemaphoreType.DMA,
        pltpu.SemaphoreType.DMA,
        pltpu.SemaphoreType.DMA,
    ],
)


def _dense_body(x_ref, to_ref, ti_ref, wz_ref, wh_ref, bz_ref, bh_ref,
                out_ref, *, relu):
    g = jnp.concatenate([x_ref[...], to_ref[...], ti_ref[...]], axis=1)
    z = jax.nn.sigmoid(
        jnp.dot(g, wz_ref[...], preferred_element_type=jnp.float32)
        + bz_ref[...])
    ht = jnp.tanh(
        jnp.dot(g, wh_ref[...], preferred_element_type=jnp.float32)
        + bh_ref[...])
    o = (1.0 - z) * ht
    if relu:
        o = jnp.maximum(o, 0.0)
    out_ref[...] = o


def _dense(x, to, ti, wz, wh, bz, bh, relu):
    blk = 2000
    grid = (N // blk,)
    row_spec = pl.BlockSpec((blk, D), lambda i: (i, 0))
    full_spec = pl.BlockSpec((3 * D, D), lambda i: (0, 0))
    bias_spec = pl.BlockSpec((1, D), lambda i: (0, 0))
    return pl.pallas_call(
        functools.partial(_dense_body, relu=relu),
        out_shape=jax.ShapeDtypeStruct((N, D), jnp.float32),
        grid=grid,
        in_specs=[row_spec, row_spec, row_spec, full_spec, full_spec,
                  bias_spec, bias_spec],
        out_specs=row_spec,
    )(x, to, ti, wz, wh, bz, bh)


def _gate_weights(W):
    a = W[0, 0][:D] + W[1, 0][:D]
    return jnp.concatenate([a, W[0, 1][:D], W[1, 1][:D]], axis=0)


def kernel(x, edge_index, edge_weight, W1z, b1z, W1r, b1r, W1h, b1h,
           W2z, b2z, W2r, b2r, W2h, b2h):
    f32 = jnp.float32
    src = edge_index[0].astype(jnp.int32)
    dst = edge_index[1].astype(jnp.int32)
    w = edge_weight.astype(f32)
    pad = EP - E
    srcp = jnp.concatenate([src, jnp.zeros((pad,), jnp.int32)])
    dstp = jnp.concatenate([dst, jnp.zeros((pad,), jnp.int32)])
    wp = jnp.concatenate([w, jnp.zeros((pad,), f32)])

    no, ni = _norms(srcp, dstp, wp)
    zrows = jnp.zeros((AZ_PER_TILE, D), f32)

    wz1 = _gate_weights(W1z)
    wh1 = _gate_weights(W1h)
    wz2 = _gate_weights(W2z)
    wh2 = _gate_weights(W2h)
    bz1 = b1z.reshape(1, D)
    bh1 = b1h.reshape(1, D)
    bz2 = b2z.reshape(1, D)
    bh2 = b2h.reshape(1, D)

    xf = x.astype(f32)
    to1, ti1 = _diffuse(zrows, xf, srcp, dstp, no, ni)
    h1 = _dense(xf, to1[:N], ti1[:N], wz1, wh1, bz1, bh1, relu=True)
    to2, ti2 = _diffuse(zrows, h1, srcp, dstp, no, ni)
    h2 = _dense(h1, to2[:N], ti2[:N], wz2, wh2, bz2, bh2, relu=False)
    return h2
